# Initial kernel scaffold; baseline (speedup 1.0000x reference)
import jax
import jax.numpy as jnp
from jax.experimental import pallas as pl


def kernel(queries, keys, values):
    raise NotImplementedError("write your pallas kernel here")


# speedup vs baseline: 2.5319x; 2.5319x over previous
import math

import numpy as np
import jax
import jax.numpy as jnp
from jax.experimental import pallas as pl

_E = 64
_TOPK = int(math.log(_E))


def _dft_mats():
    e = np.arange(_E)
    phase = 2.0 * np.pi * np.outer(e, e) / _E
    c = np.cos(phase)
    s = -np.sin(phase)
    fwd = np.concatenate([c, s], axis=1).astype(np.float32)
    inv = np.concatenate([c, s], axis=0).astype(np.float32)
    return jnp.asarray(fwd), jnp.asarray(inv)


_FWD, _INV = _dft_mats()


def _dot(a, b):
    return jax.lax.dot_general(
        a, b, (((1,), (0,)), ((), ())),
        preferred_element_type=jnp.float32,
        precision=jax.lax.Precision.HIGHEST)


def _body(q_ref, k_ref, v_ref, fwd_ref, inv_ref, v_out_ref, corr_out_ref):
    q = q_ref[0, 0]
    k = k_ref[0, 0]
    v = v_ref[0, 0]
    fwd = fwd_ref[...]
    inv = inv_ref[...]

    qf = _dot(q, fwd)
    kf = _dot(k, fwd)
    qr, qi = qf[:, :_E], qf[:, _E:]
    kr, ki = kf[:, :_E], kf[:, _E:]
    pr = qr * kr + qi * ki
    pi = qi * kr - qr * ki
    corr = _dot(jnp.concatenate([pr, pi], axis=1), inv) * (1.0 / _E)
    corr_out_ref[0] = corr.T

    iota = jax.lax.broadcasted_iota(jnp.int32, corr.shape, 1)
    work = corr
    tops, delays = [], []
    for _ in range(_TOPK):
        m = jnp.max(work, axis=1, keepdims=True)
        idx = jnp.min(jnp.where(work == m, iota, _E), axis=1, keepdims=True)
        tops.append(m)
        delays.append(idx)
        work = jnp.where(iota == idx, -jnp.inf, work)

    exps = [jnp.exp(t - tops[0]) for t in tops]
    denom = exps[0] + exps[1] + exps[2] + exps[3]

    w = jnp.zeros_like(corr)
    for i in range(_TOPK):
        w = jnp.where(iota == delays[i], exps[i] / denom, w)

    wf = _dot(w, fwd)
    vf = _dot(v, fwd)
    wr, wi = wf[:, :_E], wf[:, _E:]
    vr, vi = vf[:, :_E], vf[:, _E:]
    gr = wr * vr + wi * vi
    gi = wr * vi - wi * vr
    v_out_ref[0, 0] = _dot(jnp.concatenate([gr, gi], axis=1), inv) * (1.0 / _E)


def kernel(queries, keys, values):
    B, H, L, E = queries.shape
    lblk = 512
    grid = (B, H, L // lblk)
    row_spec = pl.BlockSpec((1, 1, lblk, E), lambda b, h, l: (b, h, l, 0))
    out_v, out_corr = pl.pallas_call(
        _body,
        grid=grid,
        in_specs=[
            row_spec, row_spec, row_spec,
            pl.BlockSpec((_E, 2 * _E), lambda b, h, l: (0, 0)),
            pl.BlockSpec((2 * _E, _E), lambda b, h, l: (0, 0)),
        ],
        out_specs=[
            row_spec,
            pl.BlockSpec((1, E, lblk), lambda b, h, l: (b, 0, h * (L // lblk) + l)),
        ],
        out_shape=[
            jax.ShapeDtypeStruct((B, H, L, E), jnp.float32),
            jax.ShapeDtypeStruct((B, E, H * L), jnp.float32),
        ],
    )(queries, keys, values, _FWD, _INV)
    return (out_v, out_corr.reshape(B, E, H, L))
